# baseline (speedup 1.0000x reference)
import functools

import jax
import jax.numpy as jnp
from jax import lax
from jax.experimental import pallas as pl
from jax.experimental.pallas import tpu as pltpu
from jax.experimental.pallas import tpu_sc as plsc

_NWORDS = 1000000
_NPAD = 1 << 20
_EMB = 64
_NTAGS = 1000
_BATCH = 4096
_HIST = 200
_TOTAL = _BATCH * _HIST

_NC = 2
_NS = 16
_NW = _NC * _NS
_PER_TILE = _TOTAL // _NW
_IDX_ROWS = _PER_TILE // 128
_NSET = 4
_ZCHUNK = 4096
_SLICE = _NPAD // _NS

_R = 16384
_NB = (_NWORDS + _R - 1) // _R


def _sc_hist_body(words_hbm, out_hbm, idx_v, ones_v, zbuf, counts_sh, sem):
    cid = lax.axis_index("c")
    sid = lax.axis_index("s")
    wid = sid * _NC + cid

    zeros16 = jnp.zeros((16,), jnp.float32)
    ones16 = jnp.full((16,), 1.0, jnp.float32)

    def fill_z(i, _):
        zbuf[pl.ds(i * 16, 16)] = zeros16
        return 0

    lax.fori_loop(0, _ZCHUNK // 16, fill_z, 0)

    def fill_o(i, _):
        ones_v[pl.ds(i * 16, 16)] = ones16
        return 0

    lax.fori_loop(0, 8, fill_o, 0)

    def zero_counts(k, _):
        pltpu.sync_copy(
            zbuf, counts_sh.at[pl.ds(sid * _SLICE + k * _ZCHUNK, _ZCHUNK)]
        )
        return 0

    lax.fori_loop(0, _SLICE // _ZCHUNK, zero_counts, 0)
    plsc.subcore_barrier()

    pltpu.sync_copy(words_hbm.at[pl.ds(wid * _IDX_ROWS, _IDX_ROWS)], idx_v)

    def scatter_group(j4, _):
        for s in range(_NSET):
            pltpu.async_copy(
                ones_v, counts_sh.at[idx_v.at[j4 * _NSET + s]], sem,
                add=True,
            )
        for s in range(_NSET):
            pltpu.make_async_copy(
                ones_v, counts_sh.at[idx_v.at[j4 * _NSET + s]], sem
            ).wait()
        return 0

    lax.fori_loop(0, _IDX_ROWS // _NSET, scatter_group, 0)
    plsc.subcore_barrier()

    pltpu.sync_copy(
        counts_sh.at[pl.ds(sid * _SLICE, _SLICE)],
        out_hbm.at[cid, pl.ds(sid * _SLICE, _SLICE)],
    )


_sc_hist = functools.partial(
    pl.kernel,
    mesh=plsc.VectorSubcoreMesh(core_axis_name="c", subcore_axis_name="s"),
    out_type=jax.ShapeDtypeStruct((_NC, _NPAD), jnp.float32),
    scratch_types=[
        pltpu.VMEM((_IDX_ROWS, 128), jnp.int32),
        pltpu.VMEM((128,), jnp.float32),
        pltpu.VMEM((_ZCHUNK,), jnp.float32),
        pltpu.VMEM_SHARED((_NPAD,), jnp.float32),
        pltpu.SemaphoreType.DMA,
    ],
)(_sc_hist_body)


def _tc_rowsum_body(t_ref, o_ref):
    g = pl.program_id(0)

    def rowsum(t):
        return jnp.sum(jnp.transpose(t), axis=0, keepdims=True)

    @pl.when(g < _NB - 1)
    def _():
        o_ref[...] = rowsum(t_ref[...])

    @pl.when(g == _NB - 1)
    def _():
        valid = _NWORDS - (_NB - 1) * _R
        rows = lax.broadcasted_iota(jnp.int32, (_R, _EMB), 0)
        o_ref[...] = rowsum(jnp.where(rows < valid, t_ref[...], 0.0))


_tc_rowsum = pl.pallas_call(
    _tc_rowsum_body,
    grid=(_NB,),
    in_specs=[pl.BlockSpec((_R, _EMB), lambda g: (g, 0))],
    out_specs=pl.BlockSpec((1, _R), lambda g: (0, g)),
    out_shape=jax.ShapeDtypeStruct((1, _NB * _R), jnp.float32),
)

_F = 65536
_FNB = _NPAD // _F


def _fin_body(c_ref, rs_ref, w0_ref, b0_ref, w1_ref, b1_ref, wout_ref,
              bout_ref, o_ref, acc):
    g = pl.program_id(0)
    c = c_ref[0:1, :] + c_ref[1:2, :]

    @pl.when(g == 0)
    def _():
        acc[0, 0] = 0.0

    @pl.when(g < _FNB - 1)
    def _():
        acc[0, 0] += jnp.sum(c * rs_ref[...])

    @pl.when(g == _FNB - 1)
    def _():
        pos = (_FNB - 1) * _F + lax.broadcasted_iota(jnp.int32, (1, _F), 1)
        acc[0, 0] += jnp.sum(jnp.where(pos < _NWORDS, c * rs_ref[...], 0.0))

        s = acc[0, 0]
        h = jnp.tanh(s * w0_ref[...] + b0_ref[...])
        h = jnp.tanh(
            lax.dot_general(
                h, w1_ref[...], (((1,), (0,)), ((), ())),
                preferred_element_type=jnp.float32,
                precision=lax.Precision.HIGHEST,
            )
            + b1_ref[...]
        )
        o_ref[...] = (
            lax.dot_general(
                h, wout_ref[...], (((1,), (0,)), ((), ())),
                preferred_element_type=jnp.float32,
                precision=lax.Precision.HIGHEST,
            )
            + bout_ref[...]
        )


_fin = pl.pallas_call(
    _fin_body,
    grid=(_FNB,),
    in_specs=[
        pl.BlockSpec((_NC, _F), lambda g: (0, g)),
        pl.BlockSpec((1, _F), lambda g: (0, g)),
        pl.BlockSpec((1, _EMB), lambda g: (0, 0)),
        pl.BlockSpec((1, _EMB), lambda g: (0, 0)),
        pl.BlockSpec((_EMB, _EMB), lambda g: (0, 0)),
        pl.BlockSpec((1, _EMB), lambda g: (0, 0)),
        pl.BlockSpec((_EMB, _NTAGS), lambda g: (0, 0)),
        pl.BlockSpec((1, _NTAGS), lambda g: (0, 0)),
    ],
    out_specs=pl.BlockSpec((1, _NTAGS), lambda g: (0, 0)),
    out_shape=jax.ShapeDtypeStruct((1, _NTAGS), jnp.float32),
    scratch_shapes=[pltpu.SMEM((1, 1), jnp.float32)],
)


def kernel(words, emb_table, W0, b0, W1, b1, Wout, bout):
    words2 = words.astype(jnp.int32).reshape(_TOTAL // 128, 128)
    counts = _sc_hist(words2)
    rowsums = _tc_rowsum(emb_table)
    rs_pad = jnp.pad(rowsums, ((0, 0), (0, _NPAD - _NB * _R)))
    return _fin(
        counts,
        rs_pad,
        W0,
        b0.reshape(1, _EMB),
        W1,
        b1.reshape(1, _EMB),
        Wout,
        bout.reshape(1, _NTAGS),
    )


# speedup vs baseline: 1.0924x; 1.0924x over previous
import functools

import jax
import jax.numpy as jnp
from jax import lax
from jax.experimental import pallas as pl
from jax.experimental.pallas import tpu as pltpu
from jax.experimental.pallas import tpu_sc as plsc

_NWORDS = 1000000
_NPAD = 1 << 20
_EMB = 64
_NTAGS = 1000
_BATCH = 4096
_HIST = 200
_TOTAL = _BATCH * _HIST

_NC = 2
_NS = 16
_NW = _NC * _NS
_PER_TILE = _TOTAL // _NW
_IDX_ROWS = _PER_TILE // 128
_NSET = 4
_ZCHUNK = 4096
_SLICE = _NPAD // _NS

_R = 32768
_NB = (_NWORDS + _R - 1) // _R


def _sc_hist_body(words_hbm, out_hbm, idx_v, ones_v, zbuf, counts_sh, sem):
    cid = lax.axis_index("c")
    sid = lax.axis_index("s")
    wid = sid * _NC + cid

    zeros16 = jnp.zeros((16,), jnp.float32)
    ones16 = jnp.full((16,), 1.0, jnp.float32)

    def fill_z(i, _):
        zbuf[pl.ds(i * 16, 16)] = zeros16
        return 0

    lax.fori_loop(0, _ZCHUNK // 16, fill_z, 0)

    def fill_o(i, _):
        ones_v[pl.ds(i * 16, 16)] = ones16
        return 0

    lax.fori_loop(0, 8, fill_o, 0)

    def zero_counts(k, _):
        pltpu.sync_copy(
            zbuf, counts_sh.at[pl.ds(sid * _SLICE + k * _ZCHUNK, _ZCHUNK)]
        )
        return 0

    lax.fori_loop(0, _SLICE // _ZCHUNK, zero_counts, 0)
    plsc.subcore_barrier()

    pltpu.sync_copy(words_hbm.at[pl.ds(wid * _IDX_ROWS, _IDX_ROWS)], idx_v)

    def scatter_group(j4, _):
        for s in range(_NSET):
            pltpu.async_copy(
                ones_v, counts_sh.at[idx_v.at[j4 * _NSET + s]], sem,
                add=True,
            )
        for s in range(_NSET):
            pltpu.make_async_copy(
                ones_v, counts_sh.at[idx_v.at[j4 * _NSET + s]], sem
            ).wait()
        return 0

    lax.fori_loop(0, _IDX_ROWS // _NSET, scatter_group, 0)
    plsc.subcore_barrier()

    pltpu.sync_copy(
        counts_sh.at[pl.ds(sid * _SLICE, _SLICE)],
        out_hbm.at[cid, pl.ds(sid * _SLICE, _SLICE)],
    )


_sc_hist = functools.partial(
    pl.kernel,
    mesh=plsc.VectorSubcoreMesh(core_axis_name="c", subcore_axis_name="s"),
    out_type=jax.ShapeDtypeStruct((_NC, _NPAD), jnp.float32),
    scratch_types=[
        pltpu.VMEM((_IDX_ROWS, 128), jnp.int32),
        pltpu.VMEM((128,), jnp.float32),
        pltpu.VMEM((_ZCHUNK,), jnp.float32),
        pltpu.VMEM_SHARED((_NPAD,), jnp.float32),
        pltpu.SemaphoreType.DMA,
    ],
)(_sc_hist_body)


def _tc_body(c_ref, t_ref, w0_ref, b0_ref, w1_ref, b1_ref, wout_ref,
             bout_ref, o_ref, acc):
    g = pl.program_id(0)
    c = c_ref[0:1, :] + c_ref[1:2, :]
    c_bf = c.astype(jnp.bfloat16)

    def _dot(a, b):
        return lax.dot_general(
            a, b, (((1,), (0,)), ((), ())),
            preferred_element_type=jnp.float32,
        )

    def contrib(t):
        t_hi = t.astype(jnp.bfloat16)
        t_lo = (t - t_hi.astype(jnp.float32)).astype(jnp.bfloat16)
        return _dot(c_bf, t_hi) + _dot(c_bf, t_lo)

    @pl.when(g == 0)
    def _():
        acc[...] = jnp.zeros((1, _EMB), jnp.float32)

    @pl.when(g < _NB - 1)
    def _():
        acc[...] += contrib(t_ref[...])

    @pl.when(g == _NB - 1)
    def _():
        valid = _NWORDS - (_NB - 1) * _R
        rows = lax.broadcasted_iota(jnp.int32, (_R, _EMB), 0)
        t = jnp.where(rows < valid, t_ref[...], 0.0)
        acc[...] += contrib(t)

        s = jnp.sum(acc[...])
        h = jnp.tanh(s * w0_ref[...] + b0_ref[...])
        h = jnp.tanh(
            lax.dot_general(
                h, w1_ref[...], (((1,), (0,)), ((), ())),
                preferred_element_type=jnp.float32,
                precision=lax.Precision.HIGHEST,
            )
            + b1_ref[...]
        )
        o_ref[...] = (
            lax.dot_general(
                h, wout_ref[...], (((1,), (0,)), ((), ())),
                preferred_element_type=jnp.float32,
                precision=lax.Precision.HIGHEST,
            )
            + bout_ref[...]
        )


_tc_reduce_mlp = pl.pallas_call(
    _tc_body,
    grid=(_NB,),
    in_specs=[
        pl.BlockSpec((_NC, _R), lambda g: (0, g)),
        pl.BlockSpec((_R, _EMB), lambda g: (g, 0)),
        pl.BlockSpec((1, _EMB), lambda g: (0, 0)),
        pl.BlockSpec((1, _EMB), lambda g: (0, 0)),
        pl.BlockSpec((_EMB, _EMB), lambda g: (0, 0)),
        pl.BlockSpec((1, _EMB), lambda g: (0, 0)),
        pl.BlockSpec((_EMB, _NTAGS), lambda g: (0, 0)),
        pl.BlockSpec((1, _NTAGS), lambda g: (0, 0)),
    ],
    out_specs=pl.BlockSpec((1, _NTAGS), lambda g: (0, 0)),
    out_shape=jax.ShapeDtypeStruct((1, _NTAGS), jnp.float32),
    scratch_shapes=[pltpu.VMEM((1, _EMB), jnp.float32)],
)


def kernel(words, emb_table, W0, b0, W1, b1, Wout, bout):
    words2 = words.astype(jnp.int32).reshape(_TOTAL // 128, 128)
    counts = _sc_hist(words2)
    return _tc_reduce_mlp(
        counts,
        emb_table,
        W0,
        b0.reshape(1, _EMB),
        W1,
        b1.reshape(1, _EMB),
        Wout,
        bout.reshape(1, _NTAGS),
    )
